# baseline (speedup 1.0000x reference)
import functools

import jax
import jax.numpy as jnp
from jax import lax
from jax.experimental import pallas as pl
from jax.experimental.pallas import tpu as pltpu
from jax.experimental.pallas import tpu_sc as plsc

VOCAB = 6
NAUX = 5
HID = 512
NC = 2
NS = 16
NW = NC * NS
ZR = 56


def _body(ids_hbm, aux16_hbm, zeros_hbm, out_hbm, idsall, a16buf, zbuf, semi, semz, sema):
    n = out_hbm.shape[0]
    rows_per_w = n // NW
    wid = lax.axis_index("s") * NC + lax.axis_index("c")
    base0 = wid * rows_per_w
    nz = rows_per_w // ZR
    rem = rows_per_w - nz * ZR

    aux_in = pltpu.async_copy(aux16_hbm.at[pl.ds(base0, rows_per_w)], a16buf, semi)
    pltpu.sync_copy(ids_hbm.at[pl.ds(base0, rows_per_w)], idsall)
    pltpu.sync_copy(zeros_hbm, zbuf)

    for k in range(nz):
        pltpu.async_copy(
            zbuf, out_hbm.at[pl.ds(base0 + k * ZR, ZR), pl.ds(16, HID - 16)], semz)
    if rem:
        pltpu.async_copy(
            zbuf.at[pl.ds(0, rem)],
            out_hbm.at[pl.ds(base0 + nz * ZR, rem), pl.ds(16, HID - 16)], semz)

    aux_in.wait()
    iota = lax.iota(jnp.int32, 16)
    ones = jnp.ones((16,), jnp.float32)

    def grp(g, carry):
        rows16 = g * 16 + iota
        idsv = plsc.load_gather(idsall, [rows16])
        plsc.store_scatter(a16buf, [rows16, idsv], ones)
        return carry

    lax.fori_loop(0, rows_per_w // 16, grp, 0)
    pltpu.async_copy(a16buf, out_hbm.at[pl.ds(base0, rows_per_w), pl.ds(0, 16)], sema)

    for k in range(nz):
        pltpu.make_async_copy(
            zbuf, out_hbm.at[pl.ds(base0, ZR), pl.ds(16, HID - 16)], semz).wait()
    if rem:
        pltpu.make_async_copy(
            zbuf.at[pl.ds(0, rem)],
            out_hbm.at[pl.ds(base0, rem), pl.ds(16, HID - 16)], semz).wait()
    pltpu.make_async_copy(
        a16buf, out_hbm.at[pl.ds(base0, rows_per_w), pl.ds(0, 16)], sema).wait()


def kernel(input_ids, aux_features):
    B, S = input_ids.shape
    N = B * S
    rows_per_w = N // NW
    ids1 = input_ids.reshape(N).astype(jnp.int32)
    aux16 = jnp.pad(aux_features.reshape(N, NAUX), ((0, 0), (VOCAB, 16 - VOCAB - NAUX)))
    zeros = jnp.zeros((ZR, HID - 16), jnp.float32)

    k = functools.partial(
        pl.kernel,
        out_type=jax.ShapeDtypeStruct((N, HID), jnp.float32),
        mesh=plsc.VectorSubcoreMesh(core_axis_name="c", subcore_axis_name="s"),
        compiler_params=pltpu.CompilerParams(needs_layout_passes=False, use_tc_tiling_on_sc=False),
        scratch_types=[
            pltpu.VMEM((rows_per_w,), jnp.int32),
            pltpu.VMEM((rows_per_w, 16), jnp.float32),
            pltpu.VMEM((ZR, HID - 16), jnp.float32),
            pltpu.SemaphoreType.DMA,
            pltpu.SemaphoreType.DMA,
            pltpu.SemaphoreType.DMA,
        ],
    )(_body)
    out = k(ids1, aux16, zeros)
    return out.reshape(B, S, HID)


# speedup vs baseline: 2.0528x; 2.0528x over previous
import functools

import jax
import jax.numpy as jnp
from jax import lax
from jax.experimental import pallas as pl
from jax.experimental.pallas import tpu as pltpu
from jax.experimental.pallas import tpu_sc as plsc

VOCAB = 6
NAUX = 5
HID = 512
NC = 2
NS = 16
NW = NC * NS
CH = 32


def _body(ids_hbm, aux16_hbm, zeros_hbm, out_hbm,
          idsall, abuf0, abuf1, buf0, buf1, semi0, semi1, semo0, semo1):
    n = out_hbm.shape[0]
    rows_per_w = n // NW
    nchunk = rows_per_w // CH
    npair = nchunk // 2
    wid = lax.axis_index("s") * NC + lax.axis_index("c")
    base0 = wid * rows_per_w
    last = n - CH

    iota = lax.iota(jnp.int32, 16)
    ones = jnp.ones((16,), jnp.float32)

    def aux_start(abuf, c, semi):
        base = jnp.minimum(base0 + c * CH, last)
        pltpu.async_copy(aux16_hbm.at[pl.ds(base, CH)], abuf, semi)

    def aux_wait(abuf, semi):
        pltpu.make_async_copy(aux16_hbm.at[pl.ds(0, CH)], abuf, semi).wait()

    def fill(buf, abuf, c):
        off = c * CH
        for row in range(CH):
            buf[row, 0:16] = abuf[row]
        for g in range(CH // 16):
            rows16 = off + g * 16 + iota
            idsv = plsc.load_gather(idsall, [rows16])
            plsc.store_scatter(buf, [g * 16 + iota, idsv], ones)

    def out_start(buf, c, semo):
        pltpu.async_copy(buf, out_hbm.at[pl.ds(base0 + c * CH, CH)], semo)

    def out_wait(buf, semo):
        pltpu.make_async_copy(buf, out_hbm.at[pl.ds(base0, CH)], semo).wait()

    aux_start(abuf0, 0, semi0)
    aux_start(abuf1, 1, semi1)
    pltpu.sync_copy(ids_hbm.at[pl.ds(base0, rows_per_w)], idsall)
    pltpu.sync_copy(zeros_hbm, buf0)
    pltpu.sync_copy(zeros_hbm, buf1)

    aux_wait(abuf0, semi0)
    fill(buf0, abuf0, 0)
    aux_start(abuf0, 2, semi0)
    out_start(buf0, 0, semo0)
    aux_wait(abuf1, semi1)
    fill(buf1, abuf1, 1)
    aux_start(abuf1, 3, semi1)
    out_start(buf1, 1, semo1)

    def pair(p, carry):
        c0 = 2 * p
        out_wait(buf0, semo0)
        aux_wait(abuf0, semi0)
        fill(buf0, abuf0, c0)
        aux_start(abuf0, c0 + 2, semi0)
        out_start(buf0, c0, semo0)
        out_wait(buf1, semo1)
        aux_wait(abuf1, semi1)
        fill(buf1, abuf1, c0 + 1)
        aux_start(abuf1, c0 + 3, semi1)
        out_start(buf1, c0 + 1, semo1)
        return carry

    lax.fori_loop(1, npair, pair, 0)
    out_wait(buf0, semo0)
    out_wait(buf1, semo1)
    aux_wait(abuf0, semi0)
    aux_wait(abuf1, semi1)


def kernel(input_ids, aux_features):
    B, S = input_ids.shape
    N = B * S
    rows_per_w = N // NW
    ids1 = input_ids.reshape(N).astype(jnp.int32)
    aux16 = jnp.pad(aux_features.reshape(N, NAUX), ((0, 0), (VOCAB, 16 - VOCAB - NAUX)))
    zeros = jnp.zeros((CH, HID), jnp.float32)

    k = functools.partial(
        pl.kernel,
        out_type=jax.ShapeDtypeStruct((N, HID), jnp.float32),
        mesh=plsc.VectorSubcoreMesh(core_axis_name="c", subcore_axis_name="s"),
        compiler_params=pltpu.CompilerParams(needs_layout_passes=False),
        scratch_types=[
            pltpu.VMEM((rows_per_w,), jnp.int32),
            pltpu.VMEM((CH, 16), jnp.float32),
            pltpu.VMEM((CH, 16), jnp.float32),
            pltpu.VMEM((CH, HID), jnp.float32),
            pltpu.VMEM((CH, HID), jnp.float32),
            pltpu.SemaphoreType.DMA,
            pltpu.SemaphoreType.DMA,
            pltpu.SemaphoreType.DMA,
            pltpu.SemaphoreType.DMA,
        ],
    )(_body)
    out = k(ids1, aux16, zeros)
    return out.reshape(B, S, HID)


# speedup vs baseline: 2.5176x; 1.2265x over previous
import functools

import jax
import jax.numpy as jnp
from jax import lax
from jax.experimental import pallas as pl
from jax.experimental.pallas import tpu as pltpu
from jax.experimental.pallas import tpu_sc as plsc

VOCAB = 6
NAUX = 5
HID = 512
NC = 2
NS = 16
NW = NC * NS
CH = 32


def _body(ids_hbm, aux_hbm, zeros_hbm, out_hbm,
          idsall, abuf0, abuf1, buf0, buf1, semi0, semi1, semo0, semo1):
    n = out_hbm.shape[0]
    rows_per_w = n // NW
    nchunk = rows_per_w // CH
    npair = nchunk // 2
    wid = lax.axis_index("s") * NC + lax.axis_index("c")
    base0 = wid * rows_per_w
    last = n - CH

    iota = lax.iota(jnp.int32, 16)
    ones = jnp.ones((16,), jnp.float32)
    zero16 = jnp.zeros((16,), jnp.float32)
    colidx = jnp.clip(iota - VOCAB, 0, NAUX - 1)
    auxmask = (iota >= VOCAB) & (iota < VOCAB + NAUX)

    def aux_start(abuf, c, semi):
        s0 = jnp.minimum(base0 + c * CH, last)
        pltpu.async_copy(aux_hbm.at[pl.ds(s0, CH)], abuf, semi)

    def aux_wait(abuf, semi):
        pltpu.make_async_copy(aux_hbm.at[pl.ds(0, CH)], abuf, semi).wait()

    def fill(buf, abuf, c):
        off = c * CH
        for row in range(CH):
            av = plsc.load_gather(abuf, [jnp.full((16,), row, jnp.int32), colidx])
            buf[row, 0:16] = jnp.where(auxmask, av, zero16)
        for g in range(CH // 16):
            rows16 = off + g * 16 + iota
            idsv = plsc.load_gather(idsall, [rows16])
            plsc.store_scatter(buf, [g * 16 + iota, idsv], ones)

    def out_start(buf, c, semo):
        pltpu.async_copy(buf, out_hbm.at[pl.ds(base0 + c * CH, CH)], semo)

    def out_wait(buf, semo):
        pltpu.make_async_copy(buf, out_hbm.at[pl.ds(base0, CH)], semo).wait()

    aux_start(abuf0, 0, semi0)
    aux_start(abuf1, 1, semi1)
    pltpu.sync_copy(ids_hbm.at[pl.ds(base0, rows_per_w)], idsall)
    pltpu.sync_copy(zeros_hbm, buf0)
    pltpu.sync_copy(zeros_hbm, buf1)

    aux_wait(abuf0, semi0)
    fill(buf0, abuf0, 0)
    aux_start(abuf0, 2, semi0)
    out_start(buf0, 0, semo0)
    aux_wait(abuf1, semi1)
    fill(buf1, abuf1, 1)
    aux_start(abuf1, 3, semi1)
    out_start(buf1, 1, semo1)

    def pair(p, carry):
        c0 = 2 * p
        out_wait(buf0, semo0)
        aux_wait(abuf0, semi0)
        fill(buf0, abuf0, c0)
        aux_start(abuf0, c0 + 2, semi0)
        out_start(buf0, c0, semo0)
        out_wait(buf1, semo1)
        aux_wait(abuf1, semi1)
        fill(buf1, abuf1, c0 + 1)
        aux_start(abuf1, c0 + 3, semi1)
        out_start(buf1, c0 + 1, semo1)
        return carry

    lax.fori_loop(1, npair, pair, 0)
    out_wait(buf0, semo0)
    out_wait(buf1, semo1)
    aux_wait(abuf0, semi0)
    aux_wait(abuf1, semi1)


def kernel(input_ids, aux_features):
    B, S = input_ids.shape
    N = B * S
    rows_per_w = N // NW
    zeros = jnp.zeros((CH, HID), jnp.float32)

    k = functools.partial(
        pl.kernel,
        out_type=jax.ShapeDtypeStruct((N, HID), jnp.float32),
        mesh=plsc.VectorSubcoreMesh(core_axis_name="c", subcore_axis_name="s"),
        compiler_params=pltpu.CompilerParams(needs_layout_passes=False),
        scratch_types=[
            pltpu.VMEM((rows_per_w,), jnp.int32),
            pltpu.VMEM((CH, NAUX), jnp.float32),
            pltpu.VMEM((CH, NAUX), jnp.float32),
            pltpu.VMEM((CH, HID), jnp.float32),
            pltpu.VMEM((CH, HID), jnp.float32),
            pltpu.SemaphoreType.DMA,
            pltpu.SemaphoreType.DMA,
            pltpu.SemaphoreType.DMA,
            pltpu.SemaphoreType.DMA,
        ],
    )(_body)
    ids1 = input_ids.reshape(N).astype(jnp.int32)
    aux2 = aux_features.reshape(N, NAUX)
    out = k(ids1, aux2, zeros)
    return out.reshape(B, S, HID)


# speedup vs baseline: 2.5200x; 1.0009x over previous
import functools

import jax
import jax.numpy as jnp
from jax import lax
from jax.experimental import pallas as pl
from jax.experimental.pallas import tpu as pltpu
from jax.experimental.pallas import tpu_sc as plsc

VOCAB = 6
NAUX = 5
HID = 512
NC = 2
NS = 16
NW = NC * NS
CH = 32


def _body(ids_hbm, aux_hbm, zeros_hbm, out_hbm,
          idsall, abuf0, abuf1, buf0, buf1, semi0, semi1, semo0, semo1):
    n = out_hbm.shape[0]
    rows_per_w = n // NW
    nchunk = rows_per_w // CH
    npair = nchunk // 2
    wid = lax.axis_index("s") * NC + lax.axis_index("c")
    base0 = wid * rows_per_w
    last = n - CH

    iota = lax.iota(jnp.int32, 16)
    ones = jnp.ones((16,), jnp.float32)
    zero16 = jnp.zeros((16,), jnp.float32)
    colidx = jnp.clip(iota - VOCAB, 0, NAUX - 1)
    auxmask = (iota >= VOCAB) & (iota < VOCAB + NAUX)

    seq = aux_hbm.shape[1]
    halves = seq // rows_per_w

    def aux_start(abuf, c, semi):
        s0 = jnp.minimum(base0 + c * CH, last)
        b = s0 // seq
        s = s0 - b * seq
        pltpu.async_copy(aux_hbm.at[b, pl.ds(s, CH), :], abuf, semi)

    def aux_wait(abuf, semi):
        pltpu.make_async_copy(aux_hbm.at[0, pl.ds(0, CH), :], abuf, semi).wait()

    def fill(buf, abuf, c):
        off = c * CH
        for row in range(CH):
            av = plsc.load_gather(abuf, [jnp.full((16,), row, jnp.int32), colidx])
            buf[row, 0:16] = jnp.where(auxmask, av, zero16)
        for g in range(CH // 16):
            rows16 = off + g * 16 + iota
            idsv = plsc.load_gather(idsall, [rows16])
            plsc.store_scatter(buf, [g * 16 + iota, idsv], ones)

    def out_start(buf, c, semo):
        pltpu.async_copy(buf, out_hbm.at[pl.ds(base0 + c * CH, CH)], semo)

    def out_wait(buf, semo):
        pltpu.make_async_copy(buf, out_hbm.at[pl.ds(base0, CH)], semo).wait()

    aux_start(abuf0, 0, semi0)
    aux_start(abuf1, 1, semi1)
    pltpu.sync_copy(ids_hbm.at[pl.ds(base0, rows_per_w)], idsall)
    pltpu.sync_copy(zeros_hbm, buf0)
    pltpu.sync_copy(zeros_hbm, buf1)

    aux_wait(abuf0, semi0)
    fill(buf0, abuf0, 0)
    aux_start(abuf0, 2, semi0)
    out_start(buf0, 0, semo0)
    aux_wait(abuf1, semi1)
    fill(buf1, abuf1, 1)
    aux_start(abuf1, 3, semi1)
    out_start(buf1, 1, semo1)

    def pair(p, carry):
        c0 = 2 * p
        out_wait(buf0, semo0)
        aux_wait(abuf0, semi0)
        fill(buf0, abuf0, c0)
        aux_start(abuf0, c0 + 2, semi0)
        out_start(buf0, c0, semo0)
        out_wait(buf1, semo1)
        aux_wait(abuf1, semi1)
        fill(buf1, abuf1, c0 + 1)
        aux_start(abuf1, c0 + 3, semi1)
        out_start(buf1, c0 + 1, semo1)
        return carry

    lax.fori_loop(1, npair, pair, 0)
    out_wait(buf0, semo0)
    out_wait(buf1, semo1)
    aux_wait(abuf0, semi0)
    aux_wait(abuf1, semi1)


def kernel(input_ids, aux_features):
    B, S = input_ids.shape
    N = B * S
    rows_per_w = N // NW
    zeros = jnp.zeros((CH, HID), jnp.float32)

    k = functools.partial(
        pl.kernel,
        out_type=jax.ShapeDtypeStruct((N, HID), jnp.float32),
        mesh=plsc.VectorSubcoreMesh(core_axis_name="c", subcore_axis_name="s"),
        compiler_params=pltpu.CompilerParams(
            needs_layout_passes=False, use_tc_tiling_on_sc=True),
        scratch_types=[
            pltpu.VMEM((rows_per_w,), jnp.int32),
            pltpu.VMEM((CH, NAUX), jnp.float32),
            pltpu.VMEM((CH, NAUX), jnp.float32),
            pltpu.VMEM((CH, HID), jnp.float32),
            pltpu.VMEM((CH, HID), jnp.float32),
            pltpu.SemaphoreType.DMA,
            pltpu.SemaphoreType.DMA,
            pltpu.SemaphoreType.DMA,
            pltpu.SemaphoreType.DMA,
        ],
    )(_body)
    ids1 = input_ids.reshape(N).astype(jnp.int32)
    out = k(ids1, aux_features, zeros)
    return out.reshape(B, S, HID)


# speedup vs baseline: 2.5760x; 1.0222x over previous
import functools

import jax
import jax.numpy as jnp
from jax import lax
from jax.experimental import pallas as pl
from jax.experimental.pallas import tpu as pltpu
from jax.experimental.pallas import tpu_sc as plsc

VOCAB = 6
NAUX = 5
HID = 512
NC = 2
NS = 16
NW = NC * NS
CH = 32


def _body(ids_hbm, aux_hbm, zeros_hbm, out_hbm,
          idsall, abuf0, abuf1, buf0, buf1, semi0, semi1, semo0, semo1):
    n = out_hbm.shape[0]
    rows_per_w = n // NW
    nchunk = rows_per_w // CH
    npair = nchunk // 2
    wid = lax.axis_index("s") * NC + lax.axis_index("c")
    base0 = wid * rows_per_w
    last = n - CH

    iota = lax.iota(jnp.int32, 16)
    ones = jnp.ones((16,), jnp.float32)
    zero16 = jnp.zeros((16,), jnp.float32)
    colidx = jnp.clip(iota - VOCAB, 0, NAUX - 1)
    auxmask = (iota >= VOCAB) & (iota < VOCAB + NAUX)

    seq = aux_hbm.shape[1]
    halves = seq // rows_per_w

    def aux_start(abuf, c, semi):
        s0 = jnp.minimum(base0 + c * CH, last)
        b = s0 // seq
        s = s0 - b * seq
        pltpu.async_copy(aux_hbm.at[b, pl.ds(s, CH), :], abuf, semi)

    def aux_wait(abuf, semi):
        pltpu.make_async_copy(aux_hbm.at[0, pl.ds(0, CH), :], abuf, semi).wait()

    def fill(buf, abuf, c):
        off = c * CH
        for row in range(CH):
            av = plsc.load_gather(abuf, [jnp.full((16,), row, jnp.int32), colidx])
            buf[row, 0:16] = jnp.where(auxmask, av, zero16)
        for g in range(CH // 16):
            rows16 = off + g * 16 + iota
            idsv = plsc.load_gather(idsall, [rows16])
            plsc.store_scatter(buf, [g * 16 + iota, idsv], ones)

    def out_start(buf, c, semo):
        pltpu.async_copy(buf, out_hbm.at[pl.ds(base0 + c * CH, CH)], semo)

    def out_wait(buf, semo):
        pltpu.make_async_copy(buf, out_hbm.at[pl.ds(base0, CH)], semo).wait()

    aux_start(abuf0, 0, semi0)
    aux_start(abuf1, 1, semi1)
    ib = base0 // seq
    pltpu.sync_copy(ids_hbm.at[ib, pl.ds(base0 - ib * seq, rows_per_w)], idsall)
    pltpu.sync_copy(zeros_hbm, buf0)
    pltpu.sync_copy(zeros_hbm, buf1)

    aux_wait(abuf0, semi0)
    fill(buf0, abuf0, 0)
    aux_start(abuf0, 2, semi0)
    out_start(buf0, 0, semo0)
    aux_wait(abuf1, semi1)
    fill(buf1, abuf1, 1)
    aux_start(abuf1, 3, semi1)
    out_start(buf1, 1, semo1)

    def pair(p, carry):
        c0 = 2 * p
        out_wait(buf0, semo0)
        aux_wait(abuf0, semi0)
        fill(buf0, abuf0, c0)
        aux_start(abuf0, c0 + 2, semi0)
        out_start(buf0, c0, semo0)
        out_wait(buf1, semo1)
        aux_wait(abuf1, semi1)
        fill(buf1, abuf1, c0 + 1)
        aux_start(abuf1, c0 + 3, semi1)
        out_start(buf1, c0 + 1, semo1)
        return carry

    lax.fori_loop(1, npair, pair, 0)
    out_wait(buf0, semo0)
    out_wait(buf1, semo1)
    aux_wait(abuf0, semi0)
    aux_wait(abuf1, semi1)


def kernel(input_ids, aux_features):
    B, S = input_ids.shape
    N = B * S
    rows_per_w = N // NW
    zeros = jnp.zeros((CH, HID), jnp.float32)

    k = functools.partial(
        pl.kernel,
        out_type=jax.ShapeDtypeStruct((N, HID), jnp.float32),
        mesh=plsc.VectorSubcoreMesh(core_axis_name="c", subcore_axis_name="s"),
        compiler_params=pltpu.CompilerParams(
            needs_layout_passes=False, use_tc_tiling_on_sc=True),
        scratch_types=[
            pltpu.VMEM((rows_per_w,), jnp.int32),
            pltpu.VMEM((CH, NAUX), jnp.float32),
            pltpu.VMEM((CH, NAUX), jnp.float32),
            pltpu.VMEM((CH, HID), jnp.float32),
            pltpu.VMEM((CH, HID), jnp.float32),
            pltpu.SemaphoreType.DMA,
            pltpu.SemaphoreType.DMA,
            pltpu.SemaphoreType.DMA,
            pltpu.SemaphoreType.DMA,
        ],
    )(_body)
    out = k(input_ids.astype(jnp.int32), aux_features, zeros)
    return out.reshape(B, S, HID)


# speedup vs baseline: 2.6795x; 1.0402x over previous
import functools

import jax
import jax.numpy as jnp
from jax import lax
from jax.experimental import pallas as pl
from jax.experimental.pallas import tpu as pltpu
from jax.experimental.pallas import tpu_sc as plsc

VOCAB = 6
NAUX = 5
HID = 512
NC = 2
NS = 16
NW = NC * NS
CH = 32


def _body(ids_hbm, aux_hbm, zeros_hbm, out_hbm,
          idsall, auxall, buf0, buf1, semo0, semo1):
    n = out_hbm.shape[0]
    rows_per_w = n // NW
    nchunk = rows_per_w // CH
    npair = nchunk // 2
    seq = ids_hbm.shape[1]
    wid = lax.axis_index("s") * NC + lax.axis_index("c")
    base0 = wid * rows_per_w
    ib = base0 // seq
    sbase = base0 - ib * seq

    iota = lax.iota(jnp.int32, 16)
    ones = jnp.ones((16,), jnp.float32)
    zero16 = jnp.zeros((16,), jnp.float32)
    colidx = jnp.clip(iota - VOCAB, 0, NAUX - 1)
    auxmask = (iota >= VOCAB) & (iota < VOCAB + NAUX)

    def fill(buf, c):
        off = c * CH
        for row in range(CH):
            avidx = jnp.full((16,), (off + row) * NAUX, jnp.int32) + colidx
            av = plsc.load_gather(auxall, [avidx])
            buf[row, 0:16] = jnp.where(auxmask, av, zero16)
        for g in range(CH // 16):
            rows16 = off + g * 16 + iota
            idsv = plsc.load_gather(idsall, [rows16])
            plsc.store_scatter(buf, [g * 16 + iota, idsv], ones)

    def out_start(buf, c, semo):
        pltpu.async_copy(buf, out_hbm.at[pl.ds(base0 + c * CH, CH)], semo)

    def out_wait(buf, semo):
        pltpu.make_async_copy(buf, out_hbm.at[pl.ds(base0, CH)], semo).wait()

    pltpu.sync_copy(ids_hbm.at[ib, pl.ds(sbase, rows_per_w)], idsall)
    pltpu.sync_copy(aux_hbm.at[ib, pl.ds(sbase * NAUX, rows_per_w * NAUX)], auxall)
    pltpu.sync_copy(zeros_hbm, buf0)
    pltpu.sync_copy(zeros_hbm, buf1)

    fill(buf0, 0)
    out_start(buf0, 0, semo0)
    fill(buf1, 1)
    out_start(buf1, 1, semo1)

    def pair(p, carry):
        c0 = 2 * p
        out_wait(buf0, semo0)
        fill(buf0, c0)
        out_start(buf0, c0, semo0)
        out_wait(buf1, semo1)
        fill(buf1, c0 + 1)
        out_start(buf1, c0 + 1, semo1)
        return carry

    lax.fori_loop(1, npair, pair, 0)
    out_wait(buf0, semo0)
    out_wait(buf1, semo1)


def kernel(input_ids, aux_features):
    B, S = input_ids.shape
    N = B * S
    rows_per_w = N // NW
    aux2d = aux_features.reshape(B, S * NAUX)
    zeros = jnp.zeros((CH, HID), jnp.float32)

    k = functools.partial(
        pl.kernel,
        out_type=jax.ShapeDtypeStruct((N, HID), jnp.float32),
        mesh=plsc.VectorSubcoreMesh(core_axis_name="c", subcore_axis_name="s"),
        compiler_params=pltpu.CompilerParams(
            needs_layout_passes=False, use_tc_tiling_on_sc=True),
        scratch_types=[
            pltpu.VMEM((rows_per_w,), jnp.int32),
            pltpu.VMEM((rows_per_w * NAUX,), jnp.float32),
            pltpu.VMEM((CH, HID), jnp.float32),
            pltpu.VMEM((CH, HID), jnp.float32),
            pltpu.SemaphoreType.DMA,
            pltpu.SemaphoreType.DMA,
        ],
    )(_body)
    out = k(input_ids.astype(jnp.int32), aux2d, zeros)
    return out.reshape(B, S, HID)
